# baseline (speedup 1.0000x reference)
import functools

import jax
import jax.numpy as jnp
from jax import lax
from jax.experimental import pallas as pl
from jax.experimental.pallas import tpu as pltpu
from jax.experimental.pallas import tpu_sc as plsc

_VOCAB = 100000
_EMBED = 64
_CTX = 50
_HID = 128
_CTX_PAD = 64

_BLK = 10000
_NBLK = 10
_NEG = -1e30


@functools.cache
def _make_sc_gather():
    @functools.partial(
        pl.kernel,
        out_type=jax.ShapeDtypeStruct((_CTX_PAD, _EMBED), jnp.float32),
        mesh=plsc.VectorSubcoreMesh(core_axis_name="c", subcore_axis_name="s"),
        scratch_types=[
            pltpu.VMEM((_CTX_PAD,), jnp.int32),
            pltpu.VMEM((_CTX_PAD, _EMBED), jnp.float32),
            pltpu.SemaphoreType.DMA,
        ],
        compiler_params=pltpu.CompilerParams(use_tc_tiling_on_sc=False),
    )
    def _sc_gather(idx_hbm, table_hbm, out_hbm, idx_v, rows_v, sem):
        cid = lax.axis_index("c")
        sid = lax.axis_index("s")

        @pl.when((cid == 0) & (sid == 0))
        def _():
            pltpu.sync_copy(idx_hbm, idx_v)
            pltpu.async_copy(table_hbm.at[idx_v], rows_v, sem).wait()
            pltpu.sync_copy(rows_v, out_hbm)

    return _sc_gather


def _mlp_body(emb_ref, w1_ref, b1_ref, w2_ref, b2_ref, out_ref,
              h_ref, vm_ref, vs_ref):
    i = pl.program_id(0)

    @pl.when(i == 0)
    def _init():
        h = lax.dot_general(emb_ref[...], w1_ref[...],
                            (((1,), (1,)), ((), ())),
                            preferred_element_type=jnp.float32)
        h_ref[...] = jnp.maximum(h + b1_ref[...], 0.0)
        vm_ref[...] = jnp.full_like(vm_ref, _NEG)
        vs_ref[...] = jnp.zeros_like(vs_ref)

    logits = lax.dot_general(h_ref[...], w2_ref[...],
                             (((1,), (1,)), ((), ())),
                             preferred_element_type=jnp.float32)
    logits = logits + b2_ref[pl.ds(i, 1), :]
    out_ref[pl.ds(i, 1), :] = logits

    bmax = jnp.max(logits, axis=1, keepdims=True)
    psum = jnp.sum(jnp.exp(logits - bmax), axis=1, keepdims=True)
    bm = jnp.broadcast_to(bmax, (1, 128))
    ps = jnp.broadcast_to(psum, (1, 128))
    vm_old = vm_ref[...]
    vm_new = jnp.maximum(vm_old, bm)
    vs_ref[...] = (vs_ref[...] * jnp.exp(vm_old - vm_new)
                   + ps * jnp.exp(bm - vm_new))
    vm_ref[...] = vm_new

    @pl.when(i == _NBLK - 1)
    def _fin():
        lse = (vm_ref[0:1, 0:1]
               + jnp.log(vs_ref[0:1, 0:1]))
        out_ref[...] = out_ref[...] - lse


def _mlp_call(embeds, W1, b1_2d, W2, b2_2d, interpret=False):
    return pl.pallas_call(
        _mlp_body,
        grid=(_NBLK,),
        in_specs=[
            pl.BlockSpec((1, _CTX * _EMBED), lambda i: (0, 0)),
            pl.BlockSpec((_HID, _CTX * _EMBED), lambda i: (0, 0)),
            pl.BlockSpec((1, _HID), lambda i: (0, 0)),
            pl.BlockSpec((_BLK, _HID), lambda i: (i, 0)),
            pl.BlockSpec((_NBLK, _BLK), lambda i: (0, 0)),
        ],
        out_specs=pl.BlockSpec((_NBLK, _BLK), lambda i: (0, 0)),
        out_shape=jax.ShapeDtypeStruct((_NBLK, _BLK), jnp.float32),
        scratch_shapes=[
            pltpu.VMEM((1, _HID), jnp.float32),
            pltpu.VMEM((1, 128), jnp.float32),
            pltpu.VMEM((1, 128), jnp.float32),
        ],
        interpret=interpret,
    )(embeds, W1, b1_2d, W2, b2_2d)


def _tc_gather_body(idx_ref, table_ref, out_ref, sem):
    copies = [
        pltpu.make_async_copy(table_ref.at[idx_ref[c]], out_ref.at[c], sem)
        for c in range(_CTX)
    ]
    for cp in copies:
        cp.start()
    for cp in copies:
        cp.wait()


def _tc_gather(idx, emb_table):
    return pl.pallas_call(
        _tc_gather_body,
        in_specs=[
            pl.BlockSpec(memory_space=pltpu.SMEM),
            pl.BlockSpec(memory_space=pl.ANY),
        ],
        out_specs=pl.BlockSpec(memory_space=pltpu.VMEM),
        out_shape=jax.ShapeDtypeStruct((_CTX, _EMBED), jnp.float32),
        scratch_shapes=[pltpu.SemaphoreType.DMA],
    )(idx, emb_table)


def kernel(inputs, emb_table, W1, b1, W2, b2):
    idx = inputs.astype(jnp.int32)
    rows = _tc_gather(idx, emb_table)
    embeds = rows.reshape(1, _CTX * _EMBED)
    out2d = _mlp_call(embeds, W1, b1.reshape(1, _HID), W2,
                      b2.reshape(_NBLK, _BLK))
    return out2d.reshape(1, _VOCAB)


# speedup vs baseline: 2.3016x; 2.3016x over previous
import jax
import jax.numpy as jnp
from jax import lax
from jax.experimental import pallas as pl
from jax.experimental.pallas import tpu as pltpu

_VOCAB = 100000
_EMBED = 64
_CTX = 50
_HID = 128

_BLK = 12800
_NBLK = 8
_PAD_N = _NBLK * _BLK
_NEG = -1e30


_WMAX = (_VOCAB - 128) // 128 * 128
_TAIL = _VOCAB - 128


def _tc_gather_body(idx_ref, tableT_ref, tail_ref, out_ref, buf_ref, sem):
    copies = []
    for c in range(_CTX):
        r = idx_ref[c]
        base = pl.multiple_of(jnp.minimum(r // 128, _WMAX // 128) * 128, 128)
        copies.append(pltpu.make_async_copy(
            tableT_ref.at[:, pl.ds(base, 128)], buf_ref.at[c], sem))
    for cp in copies:
        cp.start()
    for cp in copies:
        cp.wait()

    lane = lax.broadcasted_iota(jnp.int32, (_EMBED, 128), 1)
    tail = tail_ref[...]
    for c in range(_CTX):
        r = idx_ref[c]
        sel_main = jnp.where((r < _TAIL) & (lane == r % 128),
                             buf_ref[c], 0.0)
        sel_tail = jnp.where((r >= _TAIL) & (lane == r - _TAIL),
                             tail, 0.0)
        out_ref[:, c:c + 1] = (jnp.sum(sel_main, axis=1, keepdims=True)
                               + jnp.sum(sel_tail, axis=1, keepdims=True))


def _tc_gather(idx, tableT, tail):
    return pl.pallas_call(
        _tc_gather_body,
        in_specs=[
            pl.BlockSpec(memory_space=pltpu.SMEM),
            pl.BlockSpec(memory_space=pl.ANY),
            pl.BlockSpec(memory_space=pltpu.VMEM),
        ],
        out_specs=pl.BlockSpec(memory_space=pltpu.VMEM),
        out_shape=jax.ShapeDtypeStruct((_EMBED, _CTX), jnp.float32),
        scratch_shapes=[
            pltpu.VMEM((_CTX, _EMBED, 128), jnp.float32),
            pltpu.SemaphoreType.DMA,
        ],
    )(idx, tableT, tail)


def _mlp_body(emb_ref, w1_ref, b1_ref, w2_ref, b2_ref, out_ref,
              h_ref, vm_ref, vs_ref):
    i = pl.program_id(0)

    @pl.when(i == 0)
    def _init():
        h = lax.dot_general(emb_ref[...], w1_ref[...],
                            (((1,), (1,)), ((), ())),
                            preferred_element_type=jnp.float32)
        h_ref[...] = jnp.maximum(h + b1_ref[...], 0.0)
        vm_ref[...] = jnp.full_like(vm_ref, _NEG)
        vs_ref[...] = jnp.zeros_like(vs_ref)

    logits = lax.dot_general(h_ref[...], w2_ref[...],
                             (((1,), (1,)), ((), ())),
                             preferred_element_type=jnp.float32)
    logits = logits + b2_ref[...]
    out_ref[:, pl.ds(i * _BLK, _BLK)] = logits

    col = i * _BLK + lax.broadcasted_iota(jnp.int32, (1, _BLK), 1)
    lg = jnp.where(col < _VOCAB, logits, _NEG)
    bmax = jnp.max(lg, axis=1, keepdims=True)
    psum = jnp.sum(jnp.exp(lg - bmax), axis=1, keepdims=True)
    bm = jnp.broadcast_to(bmax, (1, 128))
    ps = jnp.broadcast_to(psum, (1, 128))
    vm_old = vm_ref[...]
    vm_new = jnp.maximum(vm_old, bm)
    vs_ref[...] = (vs_ref[...] * jnp.exp(vm_old - vm_new)
                   + ps * jnp.exp(bm - vm_new))
    vm_ref[...] = vm_new

    @pl.when(i == _NBLK - 1)
    def _fin():
        lse = (vm_ref[0:1, 0:1]
               + jnp.log(vs_ref[0:1, 0:1]))
        out_ref[...] = out_ref[...] - lse


def _mlp_call(embeds, W1, b1_2d, W2, b2_2d, interpret=False):
    return pl.pallas_call(
        _mlp_body,
        grid=(_NBLK,),
        in_specs=[
            pl.BlockSpec((1, _CTX * _EMBED), lambda i: (0, 0)),
            pl.BlockSpec((_HID, _CTX * _EMBED), lambda i: (0, 0)),
            pl.BlockSpec((1, _HID), lambda i: (0, 0)),
            pl.BlockSpec((_BLK, _HID), lambda i: (i, 0)),
            pl.BlockSpec((1, _BLK), lambda i: (0, i)),
        ],
        out_specs=pl.BlockSpec((1, _PAD_N), lambda i: (0, 0)),
        out_shape=jax.ShapeDtypeStruct((1, _VOCAB), jnp.float32),
        scratch_shapes=[
            pltpu.VMEM((1, _HID), jnp.float32),
            pltpu.VMEM((1, 128), jnp.float32),
            pltpu.VMEM((1, 128), jnp.float32),
        ],
        interpret=interpret,
    )(embeds, W1, b1_2d, W2, b2_2d)


def kernel(inputs, emb_table, W1, b1, W2, b2):
    idx = inputs.astype(jnp.int32)
    tableT = jnp.swapaxes(emb_table, 0, 1)
    tail = lax.slice(tableT, (0, _TAIL), (_EMBED, _VOCAB))
    cols = _tc_gather(idx, tableT, tail)
    embeds = jnp.swapaxes(cols, 0, 1).reshape(1, _CTX * _EMBED)
    return _mlp_call(embeds, W1, b1.reshape(1, _HID), W2,
                     b2.reshape(1, _VOCAB))


# speedup vs baseline: 2.3286x; 1.0117x over previous
import jax
import jax.numpy as jnp
from jax import lax
from jax.experimental import pallas as pl
from jax.experimental.pallas import tpu as pltpu

_VOCAB = 100000
_EMBED = 64
_CTX = 50
_HID = 128

_BLK = 25600
_NBLK = 4
_PAD_N = _NBLK * _BLK
_NEG = -1e30


_WMAX = (_VOCAB - 128) // 128 * 128
_TAIL = _VOCAB - 128


def _tc_gather_body(idx_ref, tableT_ref, tail_ref, out_ref, buf_ref, sem):
    copies = []
    for c in range(_CTX):
        r = idx_ref[c]
        base = pl.multiple_of(jnp.minimum(r // 128, _WMAX // 128) * 128, 128)
        copies.append(pltpu.make_async_copy(
            tableT_ref.at[:, pl.ds(base, 128)], buf_ref.at[c], sem))
    for cp in copies:
        cp.start()
    for cp in copies:
        cp.wait()

    lane = lax.broadcasted_iota(jnp.int32, (_EMBED, 128), 1)
    tail = tail_ref[...]
    for c in range(_CTX):
        r = idx_ref[c]
        sel_main = jnp.where((r < _TAIL) & (lane == r % 128),
                             buf_ref[c], 0.0)
        sel_tail = jnp.where((r >= _TAIL) & (lane == r - _TAIL),
                             tail, 0.0)
        out_ref[:, c:c + 1] = (jnp.sum(sel_main, axis=1, keepdims=True)
                               + jnp.sum(sel_tail, axis=1, keepdims=True))


def _tc_gather(idx, tableT, tail):
    return pl.pallas_call(
        _tc_gather_body,
        in_specs=[
            pl.BlockSpec(memory_space=pltpu.SMEM),
            pl.BlockSpec(memory_space=pl.ANY),
            pl.BlockSpec(memory_space=pltpu.VMEM),
        ],
        out_specs=pl.BlockSpec(memory_space=pltpu.VMEM),
        out_shape=jax.ShapeDtypeStruct((_EMBED, _CTX), jnp.float32),
        scratch_shapes=[
            pltpu.VMEM((_CTX, _EMBED, 128), jnp.float32),
            pltpu.SemaphoreType.DMA,
        ],
    )(idx, tableT, tail)


def _mlp_body(emb_ref, w1_ref, b1_ref, w2_ref, b2_ref, out_ref,
              h_ref, vm_ref, vs_ref):
    i = pl.program_id(0)

    @pl.when(i == 0)
    def _init():
        h = lax.dot_general(emb_ref[...], w1_ref[...],
                            (((1,), (1,)), ((), ())),
                            preferred_element_type=jnp.float32)
        h_ref[...] = jnp.maximum(h + b1_ref[...], 0.0)
        vm_ref[...] = jnp.full_like(vm_ref, _NEG)
        vs_ref[...] = jnp.zeros_like(vs_ref)

    logits = lax.dot_general(h_ref[...], w2_ref[...],
                             (((1,), (1,)), ((), ())),
                             preferred_element_type=jnp.float32)
    logits = logits + b2_ref[...]
    out_ref[:, pl.ds(i * _BLK, _BLK)] = logits

    col = i * _BLK + lax.broadcasted_iota(jnp.int32, (1, _BLK), 1)
    lg = jnp.where(col < _VOCAB, logits, _NEG)
    bmax = jnp.max(lg, axis=1, keepdims=True)
    psum = jnp.sum(jnp.exp(lg - bmax), axis=1, keepdims=True)
    bm = jnp.broadcast_to(bmax, (1, 128))
    ps = jnp.broadcast_to(psum, (1, 128))
    vm_old = vm_ref[...]
    vm_new = jnp.maximum(vm_old, bm)
    vs_ref[...] = (vs_ref[...] * jnp.exp(vm_old - vm_new)
                   + ps * jnp.exp(bm - vm_new))
    vm_ref[...] = vm_new

    @pl.when(i == _NBLK - 1)
    def _fin():
        lse = (vm_ref[0:1, 0:1]
               + jnp.log(vs_ref[0:1, 0:1]))
        out_ref[...] = out_ref[...] - lse


def _mlp_call(embeds, W1, b1_2d, W2, b2_2d, interpret=False):
    return pl.pallas_call(
        _mlp_body,
        grid=(_NBLK,),
        in_specs=[
            pl.BlockSpec((1, _CTX * _EMBED), lambda i: (0, 0)),
            pl.BlockSpec((_HID, _CTX * _EMBED), lambda i: (0, 0)),
            pl.BlockSpec((1, _HID), lambda i: (0, 0)),
            pl.BlockSpec((_BLK, _HID), lambda i: (i, 0)),
            pl.BlockSpec((1, _BLK), lambda i: (0, i)),
        ],
        out_specs=pl.BlockSpec((1, _PAD_N), lambda i: (0, 0)),
        out_shape=jax.ShapeDtypeStruct((1, _VOCAB), jnp.float32),
        scratch_shapes=[
            pltpu.VMEM((1, _HID), jnp.float32),
            pltpu.VMEM((1, 128), jnp.float32),
            pltpu.VMEM((1, 128), jnp.float32),
        ],
        interpret=interpret,
    )(embeds, W1, b1_2d, W2, b2_2d)


def kernel(inputs, emb_table, W1, b1, W2, b2):
    idx = inputs.astype(jnp.int32)
    tableT = jnp.swapaxes(emb_table, 0, 1)
    tail = lax.slice(tableT, (0, _TAIL), (_EMBED, _VOCAB))
    cols = _tc_gather(idx, tableT, tail)
    embeds = jnp.swapaxes(cols, 0, 1).reshape(1, _CTX * _EMBED)
    return _mlp_call(embeds, W1, b1.reshape(1, _HID), W2,
                     b2.reshape(1, _VOCAB))


# speedup vs baseline: 2.4534x; 1.0536x over previous
import jax
import jax.numpy as jnp
from jax import lax
from jax.experimental import pallas as pl
from jax.experimental.pallas import tpu as pltpu

_VOCAB = 100000
_EMBED = 64
_CTX = 50
_HID = 128

_BLK = 25600
_NBLK = 4
_PAD_N = _NBLK * _BLK
_NEG = -1e30


_WMAX = (_VOCAB - 128) // 128 * 128
_TWID = _VOCAB - _WMAX
_TCUT = _WMAX + 128


def _tc_gather_body(idx_ref, tableT_ref, tail_ref, out_ref, buf_ref, sem):
    copies = []
    for c in range(_CTX):
        r = idx_ref[c]
        base = pl.multiple_of(jnp.minimum(r // 128, _WMAX // 128) * 128, 128)
        copies.append(pltpu.make_async_copy(
            tableT_ref.at[:, pl.ds(base, 128)], buf_ref.at[c], sem))
    for cp in copies:
        cp.start()
    for cp in copies:
        cp.wait()

    lane = lax.broadcasted_iota(jnp.int32, (_EMBED, 128), 1)
    lane_t = lax.broadcasted_iota(jnp.int32, (_EMBED, _TWID), 1)
    tail = tail_ref[...]
    for c in range(_CTX):
        r = idx_ref[c]
        sel_main = jnp.where((r < _TCUT) & (lane == r % 128),
                             buf_ref[c], 0.0)
        sel_tail = jnp.where((r >= _TCUT) & (lane_t == r - _WMAX),
                             tail, 0.0)
        out_ref[:, c:c + 1] = (jnp.sum(sel_main, axis=1, keepdims=True)
                               + jnp.sum(sel_tail, axis=1, keepdims=True))


def _tc_gather(idx, tableT, tail):
    return pl.pallas_call(
        _tc_gather_body,
        in_specs=[
            pl.BlockSpec(memory_space=pltpu.SMEM),
            pl.BlockSpec(memory_space=pl.ANY),
            pl.BlockSpec(memory_space=pltpu.VMEM),
        ],
        out_specs=pl.BlockSpec(memory_space=pltpu.VMEM),
        out_shape=jax.ShapeDtypeStruct((_EMBED, _CTX), jnp.float32),
        scratch_shapes=[
            pltpu.VMEM((_CTX, _EMBED, 128), jnp.float32),
            pltpu.SemaphoreType.DMA,
        ],
    )(idx, tableT, tail)


def _mlp_body(emb_ref, w1_ref, b1_ref, w2_ref, b2_ref, out_ref,
              h_ref, vm_ref, vs_ref):
    i = pl.program_id(0)

    @pl.when(i == 0)
    def _init():
        h = lax.dot_general(emb_ref[...], w1_ref[...],
                            (((1,), (1,)), ((), ())),
                            preferred_element_type=jnp.float32)
        h_ref[...] = jnp.maximum(h + b1_ref[...], 0.0)
        vm_ref[...] = jnp.full_like(vm_ref, _NEG)
        vs_ref[...] = jnp.zeros_like(vs_ref)

    logits = lax.dot_general(h_ref[...], w2_ref[...],
                             (((1,), (1,)), ((), ())),
                             preferred_element_type=jnp.float32)
    logits = logits + b2_ref[...][None, :]
    out_ref[:, pl.ds(i * _BLK, _BLK)] = logits

    col = i * _BLK + lax.broadcasted_iota(jnp.int32, (1, _BLK), 1)
    lg = jnp.where(col < _VOCAB, logits, _NEG)
    bmax = jnp.max(lg, axis=1, keepdims=True)
    psum = jnp.sum(jnp.exp(lg - bmax), axis=1, keepdims=True)
    bm = jnp.broadcast_to(bmax, (1, 128))
    ps = jnp.broadcast_to(psum, (1, 128))
    vm_old = vm_ref[...]
    vm_new = jnp.maximum(vm_old, bm)
    vs_ref[...] = (vs_ref[...] * jnp.exp(vm_old - vm_new)
                   + ps * jnp.exp(bm - vm_new))
    vm_ref[...] = vm_new

    @pl.when(i == _NBLK - 1)
    def _fin():
        lse = (vm_ref[0:1, 0:1]
               + jnp.log(vs_ref[0:1, 0:1]))
        out_ref[...] = out_ref[...] - lse


def _mlp_call(embeds, W1, b1_2d, W2, b2_1d, interpret=False):
    return pl.pallas_call(
        _mlp_body,
        grid=(_NBLK,),
        in_specs=[
            pl.BlockSpec((1, _CTX * _EMBED), lambda i: (0, 0)),
            pl.BlockSpec((_HID, _CTX * _EMBED), lambda i: (0, 0)),
            pl.BlockSpec((1, _HID), lambda i: (0, 0)),
            pl.BlockSpec((_BLK, _HID), lambda i: (i, 0)),
            pl.BlockSpec((_BLK,), lambda i: (i,)),
        ],
        out_specs=pl.BlockSpec((1, _PAD_N), lambda i: (0, 0)),
        out_shape=jax.ShapeDtypeStruct((1, _VOCAB), jnp.float32),
        scratch_shapes=[
            pltpu.VMEM((1, _HID), jnp.float32),
            pltpu.VMEM((1, 128), jnp.float32),
            pltpu.VMEM((1, 128), jnp.float32),
        ],
        interpret=interpret,
    )(embeds, W1, b1_2d, W2, b2_1d)


def kernel(inputs, emb_table, W1, b1, W2, b2):
    idx = inputs.astype(jnp.int32)
    tableT = jnp.swapaxes(emb_table, 0, 1)
    tail = lax.slice(tableT, (0, _WMAX), (_EMBED, _VOCAB))
    cols = _tc_gather(idx, tableT, tail)
    embeds = jnp.swapaxes(cols, 0, 1).reshape(1, _CTX * _EMBED)
    return _mlp_call(embeds, W1, b1.reshape(1, _HID), W2, b2)
